# baseline (speedup 1.0000x reference)
import functools
import jax
import jax.numpy as jnp
from jax.experimental import pallas as pl
from jax.experimental.pallas import tpu as pltpu

B, S, D, H, DH, L, R, BL, DFF = 1, 4096, 768, 12, 64, 2, 4, 64, 3072
NB = S // BL
NC = S // BL
N = B * H
NR = N * R
CS = 512
CB = 4

_PREC = jax.lax.Precision.HIGHEST


def _dot(a, b, prec=_PREC):
    return jnp.dot(a, b, preferred_element_type=jnp.float32, precision=prec)


def _pre_body(x_ref, g_ref, b_ref, wqk_ref, wv_ref, qk_ref, v_ref):
    x = x_ref[...]
    m = jnp.mean(x, axis=-1, keepdims=True)
    var = jnp.mean((x - m) ** 2, axis=-1, keepdims=True)
    ln = (x - m) / jnp.sqrt(var + 1e-5) * g_ref[...] + b_ref[...]
    qk_ref[...] = _dot(ln, wqk_ref[...])
    v_ref[...] = _dot(ln, wv_ref[...])


def _pre_call(x2, g, b, wqk, wv):
    grid = (S // CS,)
    return pl.pallas_call(
        _pre_body,
        grid=grid,
        in_specs=[
            pl.BlockSpec((CS, D), lambda i: (i, 0)),
            pl.BlockSpec((1, D), lambda i: (0, 0)),
            pl.BlockSpec((1, D), lambda i: (0, 0)),
            pl.BlockSpec((D, D), lambda i: (0, 0)),
            pl.BlockSpec((D, D), lambda i: (0, 0)),
        ],
        out_specs=[
            pl.BlockSpec((CS, D), lambda i: (i, 0)),
            pl.BlockSpec((CS, D), lambda i: (i, 0)),
        ],
        out_shape=[
            jax.ShapeDtypeStruct((S, D), jnp.float32),
            jax.ShapeDtypeStruct((S, D), jnp.float32),
        ],
    )(x2, g.reshape(1, D), b.reshape(1, D), wqk, wv)


def _attn_body(q_ref, kp_ref, vc_ref, vp_ref, qps_ref, kpl_ref, kpe_ref,
               so_ref, slog_ref):
    for j in range(CB):
        q = q_ref[0, j]
        kcur = q_ref[0, j]
        kprev = kp_ref[0, 0] if j == 0 else q_ref[0, j - 1]
        k = jnp.concatenate([kprev, kcur], axis=0)
        kn = k / (jnp.sqrt(jnp.sum(k * k, axis=-1, keepdims=True)) + 1e-9)
        vcur = vc_ref[0, j]
        vprev = vp_ref[0, 0] if j == 0 else vc_ref[0, j - 1]
        v = jnp.concatenate([vprev, vcur], axis=0)
        qp = qps_ref[0, j]
        kpprev = kpe_ref[0, 0] if j == 0 else kpl_ref[0, j - 1]
        kp = jnp.concatenate([kpprev, kpl_ref[0, j]], axis=-1)
        dots = jax.lax.dot_general(
            q, kn, (((1,), (1,)), ((), ())),
            preferred_element_type=jnp.float32, precision=_PREC) * 0.125
        dots = jnp.where(qp < kp, -1e9, dots)
        dots = jnp.where(qp == kp, dots - 1e5, dots)
        mx = jnp.max(dots, axis=-1, keepdims=True)
        ex = jnp.exp(dots - mx)
        sm = jnp.sum(ex, axis=-1, keepdims=True)
        num = jax.lax.dot_general(
            ex, v, (((1,), (0,)), ((), ())),
            preferred_element_type=jnp.float32, precision=_PREC)
        so_ref[0, j] = num / sm
        slog_ref[0, j] = mx + jnp.log(sm)


def _attn_call(sqk4, sv4, pos_s, pos_l):
    grid = (NR, NC // CB)
    prev_edge = lambda n, g: (n, (g * CB + NC - 1) % NC, 0, 0)
    return pl.pallas_call(
        _attn_body,
        grid=grid,
        in_specs=[
            pl.BlockSpec((1, CB, BL, DH), lambda n, g: (n, g, 0, 0)),
            pl.BlockSpec((1, 1, BL, DH), prev_edge),
            pl.BlockSpec((1, CB, BL, DH), lambda n, g: (n, g, 0, 0)),
            pl.BlockSpec((1, 1, BL, DH), prev_edge),
            pl.BlockSpec((1, CB, BL, 1), lambda n, g: (n, g, 0, 0)),
            pl.BlockSpec((1, CB, 1, BL), lambda n, g: (n, g, 0, 0)),
            pl.BlockSpec((1, 1, 1, BL), prev_edge),
        ],
        out_specs=[
            pl.BlockSpec((1, CB, BL, DH), lambda n, g: (n, g, 0, 0)),
            pl.BlockSpec((1, CB, BL, 1), lambda n, g: (n, g, 0, 0)),
        ],
        out_shape=[
            jax.ShapeDtypeStruct((NR, NC, BL, DH), jnp.float32),
            jax.ShapeDtypeStruct((NR, NC, BL, 1), jnp.float32),
        ],
    )(sqk4, sqk4, sv4, sv4, pos_s, pos_l, pos_l)


CSC = 256


def _comb_body(o_ref, lg_ref, x1_ref, wo_ref, y1_ref):
    outs = []
    for h in range(H):
        lgs = [lg_ref[h * R + r] for r in range(R)]
        m = jnp.maximum(jnp.maximum(lgs[0], lgs[1]),
                        jnp.maximum(lgs[2], lgs[3]))
        es = [jnp.exp(l - m) for l in lgs]
        den = es[0] + es[1] + es[2] + es[3]
        acc = o_ref[h * R] * (es[0] / den)
        for r in range(1, R):
            acc = acc + o_ref[h * R + r] * (es[r] / den)
        outs.append(acc)
    out = jnp.concatenate(outs, axis=-1)
    y1_ref[...] = x1_ref[...] + _dot(out, wo_ref[...])


def _comb_call(o_u, lg_u, x1, wo):
    grid = (S // CSC,)
    return pl.pallas_call(
        _comb_body,
        grid=grid,
        in_specs=[
            pl.BlockSpec((NR, CSC, DH), lambda i: (0, i, 0)),
            pl.BlockSpec((NR, CSC, 1), lambda i: (0, i, 0)),
            pl.BlockSpec((CSC, D), lambda i: (i, 0)),
            pl.BlockSpec((D, D), lambda i: (0, 0)),
        ],
        out_specs=pl.BlockSpec((CSC, D), lambda i: (i, 0)),
        out_shape=jax.ShapeDtypeStruct((S, D), jnp.float32),
    )(o_u, lg_u, x1, wo)


def _ffn_body(y1_ref, x2_ref, g2_ref, b2g_ref, w1_ref, b1_ref, w2_ref,
              b2_ref, y2_ref):
    y1 = y1_ref[...]
    m = jnp.mean(y1, axis=-1, keepdims=True)
    var = jnp.mean((y1 - m) ** 2, axis=-1, keepdims=True)
    ln = (y1 - m) / jnp.sqrt(var + 1e-5) * g2_ref[...] + b2g_ref[...]
    hdn = jnp.maximum(_dot(ln, w1_ref[...]) + b1_ref[...], 0.0)
    f = _dot(hdn, w2_ref[...]) + b2_ref[...]
    y2_ref[...] = x2_ref[...] + f


def _ffn_call(y1, x2, g2, b2g, w1, b1, w2, b2):
    grid = (S // CS,)
    return pl.pallas_call(
        _ffn_body,
        grid=grid,
        in_specs=[
            pl.BlockSpec((CS, D), lambda i: (i, 0)),
            pl.BlockSpec((CS, D), lambda i: (i, 0)),
            pl.BlockSpec((1, D), lambda i: (0, 0)),
            pl.BlockSpec((1, D), lambda i: (0, 0)),
            pl.BlockSpec((D, DFF), lambda i: (0, 0)),
            pl.BlockSpec((1, DFF), lambda i: (0, 0)),
            pl.BlockSpec((DFF, D), lambda i: (0, 0)),
            pl.BlockSpec((1, D), lambda i: (0, 0)),
        ],
        out_specs=pl.BlockSpec((CS, D), lambda i: (i, 0)),
        out_shape=jax.ShapeDtypeStruct((S, D), jnp.float32),
    )(y1, x2, g2.reshape(1, D), b2g.reshape(1, D),
      w1, b1.reshape(1, DFF), w2, b2.reshape(1, D))


def _buckets_for_layer(x2, g, b, wqk, rot):
    xm = x2.mean(-1, keepdims=True)
    xv = ((x2 - xm) ** 2).mean(-1, keepdims=True)
    x = (x2 - xm) / jnp.sqrt(xv + 1e-5) * g + b
    qk = (x @ wqk).reshape(S, H, DH).transpose(1, 0, 2)
    rotated = jnp.einsum('nsd,rdb->nrsb', qk, rot)
    rotated = jnp.concatenate([rotated, -rotated], axis=-1)
    return jnp.argmax(rotated, axis=-1).reshape(NR, S)


def _layer(x1, x2, wqk, wv, wo, g1, b1g, w1, b1, w2, b2, g2, b2g, rot):
    qk_f, v_f = _pre_call(x2, g1, b1g, wqk, wv)
    qk_h = qk_f.reshape(S, H, DH).transpose(1, 0, 2)
    v_h = v_f.reshape(S, H, DH).transpose(1, 0, 2)

    buckets = _buckets_for_layer(x2, g1, b1g, wqk, rot)
    tick = jnp.arange(S)
    oh = (buckets[..., None] == jnp.arange(NB)).astype(jnp.float32)
    cum = jnp.cumsum(oh, axis=1)
    rank = jnp.sum(cum * oh, axis=-1) - 1.0
    totals = cum[:, -1, :]
    offs = jnp.cumsum(totals, axis=-1) - totals
    off_i = jnp.sum(offs[:, None, :] * oh, axis=-1)
    undo = (off_i + rank).astype(jnp.int32)
    rows = jnp.arange(NR)[:, None]
    sticker = jnp.zeros((NR, S), jnp.int32).at[rows, undo].set(tick[None, :])

    qk_r = jnp.repeat(qk_h, R, axis=0)
    v_r = jnp.repeat(v_h, R, axis=0)
    sqk = jnp.take_along_axis(qk_r, sticker[..., None], axis=1)
    sv = jnp.take_along_axis(v_r, sticker[..., None], axis=1)

    sqk4 = sqk.reshape(NR, NC, BL, DH)
    sv4 = sv.reshape(NR, NC, BL, DH)
    stick3 = sticker.reshape(NR, NC, BL).astype(jnp.float32)
    pos_s = stick3[..., None]
    pos_l = stick3[:, :, None, :]

    so4, slog4 = _attn_call(sqk4, sv4, pos_s, pos_l)
    so = so4.reshape(NR, S, DH)
    slog = slog4.reshape(NR, S)
    o_u = jnp.take_along_axis(so, undo[..., None], axis=1)
    lg_u = jnp.take_along_axis(slog, undo, axis=1)[..., None]

    y1 = _comb_call(o_u, lg_u, x1, wo)
    y2 = _ffn_call(y1, x2, g2, b2g, w1, b1, w2, b2)
    return y1, y2


def kernel(x1, x2, mask, Wqk, Wv, Wo, ln1_g, ln1_b, W1, b1, W2, b2,
           ln2_g, ln2_b, rot):
    a1 = x1[0]
    a2 = x2[0]
    for l in range(L):
        y1, y2 = _layer(a1, a2, Wqk[l], Wv[l], Wo[l], ln1_g[l], ln1_b[l],
                        W1[l], b1[l], W2[l], b2[l], ln2_g[l], ln2_b[l],
                        rot[l])
        a1, a2 = y1, y2
    return a2[None]


# speedup vs baseline: 2.0752x; 2.0752x over previous
import functools
import jax
import jax.numpy as jnp
from jax import lax
from jax.experimental import pallas as pl
from jax.experimental.pallas import tpu as pltpu
from jax.experimental.pallas import tpu_sc as plsc

B, S, D, H, DH, L, R, BL, DFF = 1, 4096, 768, 12, 64, 2, 4, 64, 3072
NB = S // BL
NC = S // BL
N = B * H
NR = N * R
CS = 512
CB = 4

_PREC = jax.lax.Precision.HIGHEST


def _dot(a, b, prec=_PREC):
    return jnp.dot(a, b, preferred_element_type=jnp.float32, precision=prec)


def _pre_body(x_ref, g_ref, b_ref, wqk_ref, wv_ref, qv_ref):
    x = x_ref[...]
    m = jnp.mean(x, axis=-1, keepdims=True)
    var = jnp.mean((x - m) ** 2, axis=-1, keepdims=True)
    ln = (x - m) / jnp.sqrt(var + 1e-5) * g_ref[...] + b_ref[...]
    qk = _dot(ln, wqk_ref[...])
    v = _dot(ln, wv_ref[...])
    for h in range(H):
        qv_ref[h, :, :DH] = qk[:, h * DH:(h + 1) * DH]
        qv_ref[h, :, DH:] = v[:, h * DH:(h + 1) * DH]


def _pre_call(x2, g, b, wqk, wv):
    grid = (S // CS,)
    return pl.pallas_call(
        _pre_body,
        grid=grid,
        in_specs=[
            pl.BlockSpec((CS, D), lambda i: (i, 0)),
            pl.BlockSpec((1, D), lambda i: (0, 0)),
            pl.BlockSpec((1, D), lambda i: (0, 0)),
            pl.BlockSpec((D, D), lambda i: (0, 0)),
            pl.BlockSpec((D, D), lambda i: (0, 0)),
        ],
        out_specs=pl.BlockSpec((H, CS, 2 * DH), lambda i: (0, i, 0)),
        out_shape=jax.ShapeDtypeStruct((H, S, 2 * DH), jnp.float32),
    )(x2, g.reshape(1, D), b.reshape(1, D), wqk, wv)


def _attn_body(qv_ref, qvp_ref, qps_ref, kpl_ref, kpe_ref, solg_ref):
    for j in range(CB):
        blk = qv_ref[0, j]
        pblk = qvp_ref[0, 0] if j == 0 else qv_ref[0, j - 1]
        q = blk[:, :DH]
        k = jnp.concatenate([pblk[:, :DH], blk[:, :DH]], axis=0)
        kn = k / (jnp.sqrt(jnp.sum(k * k, axis=-1, keepdims=True)) + 1e-9)
        v = jnp.concatenate([pblk[:, DH:], blk[:, DH:]], axis=0)
        qp = qps_ref[0, j]
        kpprev = kpe_ref[0, 0] if j == 0 else kpl_ref[0, j - 1]
        kp = jnp.concatenate([kpprev, kpl_ref[0, j]], axis=-1)
        dots = jax.lax.dot_general(
            q, kn, (((1,), (1,)), ((), ())),
            preferred_element_type=jnp.float32, precision=_PREC) * 0.125
        dots = jnp.where(qp < kp, -1e9, dots)
        dots = jnp.where(qp == kp, dots - 1e5, dots)
        mx = jnp.max(dots, axis=-1, keepdims=True)
        ex = jnp.exp(dots - mx)
        sm = jnp.sum(ex, axis=-1, keepdims=True)
        num = jax.lax.dot_general(
            ex, v, (((1,), (0,)), ((), ())),
            preferred_element_type=jnp.float32, precision=_PREC)
        lg = mx + jnp.log(sm)
        solg_ref[0, j] = jnp.concatenate(
            [num / sm, jnp.broadcast_to(lg, (BL, DH))], axis=-1)


def _attn_call(qv4, pos_s, pos_l):
    grid = (NR, NC // CB)
    prev_edge = lambda n, g: (n, (g * CB + NC - 1) % NC, 0, 0)
    return pl.pallas_call(
        _attn_body,
        grid=grid,
        in_specs=[
            pl.BlockSpec((1, CB, BL, 2 * DH), lambda n, g: (n, g, 0, 0)),
            pl.BlockSpec((1, 1, BL, 2 * DH), prev_edge),
            pl.BlockSpec((1, CB, BL, 1), lambda n, g: (n, g, 0, 0)),
            pl.BlockSpec((1, CB, 1, BL), lambda n, g: (n, g, 0, 0)),
            pl.BlockSpec((1, 1, 1, BL), prev_edge),
        ],
        out_specs=pl.BlockSpec((1, CB, BL, 2 * DH), lambda n, g: (n, g, 0, 0)),
        out_shape=jax.ShapeDtypeStruct((NR, NC, BL, 2 * DH), jnp.float32),
    )(qv4, qv4, pos_s, pos_l, pos_l)


CSC = 256


def _comb_body(olg_ref, x1_ref, wo_ref, y1_ref):
    outs = []
    for h in range(H):
        blks = [olg_ref[h * R + r] for r in range(R)]
        lgs = [b[:, DH:DH + 1] for b in blks]
        m = jnp.maximum(jnp.maximum(lgs[0], lgs[1]),
                        jnp.maximum(lgs[2], lgs[3]))
        es = [jnp.exp(l - m) for l in lgs]
        den = es[0] + es[1] + es[2] + es[3]
        acc = blks[0][:, :DH] * (es[0] / den)
        for r in range(1, R):
            acc = acc + blks[r][:, :DH] * (es[r] / den)
        outs.append(acc)
    out = jnp.concatenate(outs, axis=-1)
    y1_ref[...] = x1_ref[...] + _dot(out, wo_ref[...])


def _comb_call(olg_u, x1, wo):
    grid = (S // CSC,)
    return pl.pallas_call(
        _comb_body,
        grid=grid,
        in_specs=[
            pl.BlockSpec((NR, CSC, 2 * DH), lambda i: (0, i, 0)),
            pl.BlockSpec((CSC, D), lambda i: (i, 0)),
            pl.BlockSpec((D, D), lambda i: (0, 0)),
        ],
        out_specs=pl.BlockSpec((CSC, D), lambda i: (i, 0)),
        out_shape=jax.ShapeDtypeStruct((S, D), jnp.float32),
    )(olg_u, x1, wo)


def _ffn_body(y1_ref, x2_ref, g2_ref, b2g_ref, w1_ref, b1_ref, w2_ref,
              b2_ref, y2_ref):
    y1 = y1_ref[...]
    m = jnp.mean(y1, axis=-1, keepdims=True)
    var = jnp.mean((y1 - m) ** 2, axis=-1, keepdims=True)
    ln = (y1 - m) / jnp.sqrt(var + 1e-5) * g2_ref[...] + b2g_ref[...]
    hdn = jnp.maximum(_dot(ln, w1_ref[...]) + b1_ref[...], 0.0)
    f = _dot(hdn, w2_ref[...]) + b2_ref[...]
    y2_ref[...] = x2_ref[...] + f


def _ffn_call(y1, x2, g2, b2g, w1, b1, w2, b2):
    grid = (S // CS,)
    return pl.pallas_call(
        _ffn_body,
        grid=grid,
        in_specs=[
            pl.BlockSpec((CS, D), lambda i: (i, 0)),
            pl.BlockSpec((CS, D), lambda i: (i, 0)),
            pl.BlockSpec((1, D), lambda i: (0, 0)),
            pl.BlockSpec((1, D), lambda i: (0, 0)),
            pl.BlockSpec((D, DFF), lambda i: (0, 0)),
            pl.BlockSpec((1, DFF), lambda i: (0, 0)),
            pl.BlockSpec((DFF, D), lambda i: (0, 0)),
            pl.BlockSpec((1, D), lambda i: (0, 0)),
        ],
        out_specs=pl.BlockSpec((CS, D), lambda i: (i, 0)),
        out_shape=jax.ShapeDtypeStruct((S, D), jnp.float32),
    )(y1, x2, g2.reshape(1, D), b2g.reshape(1, D),
      w1, b1.reshape(1, DFF), w2, b2.reshape(1, D))


_NW = 32
_GC = 512


def _sc_gather_rows(idx, src):
    def body(idx_ref, src_ref, out_hbm, ivm, rows, sem):
        wid = lax.axis_index("s") * 2 + lax.axis_index("c")
        for t in range(2):
            nr = wid + _NW * t

            @pl.when(nr < NR)
            def _():
                def g_body(g, carry):
                    pltpu.sync_copy(idx_ref.at[nr, pl.ds(g * _GC, _GC)], ivm)
                    pltpu.async_copy(src_ref.at[ivm], rows, sem).wait()
                    pltpu.sync_copy(rows,
                                    out_hbm.at[nr, pl.ds(g * _GC, _GC)])
                    return carry

                lax.fori_loop(0, S // _GC, g_body, 0)

    f = pl.kernel(
        body,
        mesh=plsc.VectorSubcoreMesh(core_axis_name="c", subcore_axis_name="s"),
        out_type=jax.ShapeDtypeStruct((NR, S, 2 * DH), jnp.float32),
        scratch_types=[
            pltpu.VMEM((_GC,), jnp.int32),
            pltpu.VMEM((_GC, 2 * DH), jnp.float32),
            pltpu.SemaphoreType.DMA,
        ],
    )
    return f(idx, src)


def _buckets_for_layer(x2, g, b, wqk, rot):
    xm = x2.mean(-1, keepdims=True)
    xv = ((x2 - xm) ** 2).mean(-1, keepdims=True)
    x = (x2 - xm) / jnp.sqrt(xv + 1e-5) * g + b
    qk = (x @ wqk).reshape(S, H, DH).transpose(1, 0, 2)
    rotated = jnp.einsum('nsd,rdb->nrsb', qk, rot)
    rotated = jnp.concatenate([rotated, -rotated], axis=-1)
    return jnp.argmax(rotated, axis=-1).reshape(NR, S)


def _layer(x1, x2, wqk, wv, wo, g1, b1g, w1, b1, w2, b2, g2, b2g, rot):
    qv_h = _pre_call(x2, g1, b1g, wqk, wv)

    buckets = _buckets_for_layer(x2, g1, b1g, wqk, rot)
    tick = jnp.arange(S)
    sticker = jnp.argsort(buckets * S + tick[None, :], axis=-1)
    undo = jnp.argsort(sticker, axis=-1)
    row_off = (jnp.arange(NR, dtype=jnp.int32) // R * S)[:, None]

    sqv = _sc_gather_rows(sticker + row_off, qv_h.reshape(N * S, 2 * DH))

    qv4 = sqv.reshape(NR, NC, BL, 2 * DH)
    stick3 = sticker.reshape(NR, NC, BL).astype(jnp.float32)
    pos_s = stick3[..., None]
    pos_l = stick3[:, :, None, :]

    solg4 = _attn_call(qv4, pos_s, pos_l)
    row_off2 = (jnp.arange(NR, dtype=jnp.int32) * S)[:, None]
    olg_u = _sc_gather_rows(undo + row_off2,
                            solg4.reshape(NR * S, 2 * DH))

    y1 = _comb_call(olg_u, x1, wo)
    y2 = _ffn_call(y1, x2, g2, b2g, w1, b1, w2, b2)
    return y1, y2


def kernel(x1, x2, mask, Wqk, Wv, Wo, ln1_g, ln1_b, W1, b1, W2, b2,
           ln2_g, ln2_b, rot):
    a1 = x1[0]
    a2 = x2[0]
    for l in range(L):
        y1, y2 = _layer(a1, a2, Wqk[l], Wv[l], Wo[l], ln1_g[l], ln1_b[l],
                        W1[l], b1[l], W2[l], b2[l], ln2_g[l], ln2_b[l],
                        rot[l])
        a1, a2 = y1, y2
    return a2[None]


# speedup vs baseline: 2.5796x; 1.2431x over previous
import functools
import jax
import jax.numpy as jnp
from jax import lax
from jax.experimental import pallas as pl
from jax.experimental.pallas import tpu as pltpu
from jax.experimental.pallas import tpu_sc as plsc

B, S, D, H, DH, L, R, BL, DFF = 1, 4096, 768, 12, 64, 2, 4, 64, 3072
NB = S // BL
NC = S // BL
N = B * H
NR = N * R
CS = 512
CB = 4

_PREC = jax.lax.Precision.HIGHEST


def _dot(a, b, prec=_PREC):
    return jnp.dot(a, b, preferred_element_type=jnp.float32, precision=prec)


def _pre_body(prec, x_ref, g_ref, b_ref, wqk_ref, wv_ref, qv_ref):
    x = x_ref[...]
    m = jnp.mean(x, axis=-1, keepdims=True)
    var = jnp.mean((x - m) ** 2, axis=-1, keepdims=True)
    ln = (x - m) / jnp.sqrt(var + 1e-5) * g_ref[...] + b_ref[...]
    qk = _dot(ln, wqk_ref[...], prec)
    v = _dot(ln, wv_ref[...], prec)
    for h in range(H):
        qv_ref[h, :, :DH] = qk[:, h * DH:(h + 1) * DH]
        qv_ref[h, :, DH:] = v[:, h * DH:(h + 1) * DH]


def _pre_call(x2, g, b, wqk, wv, prec):
    grid = (S // CS,)
    return pl.pallas_call(
        functools.partial(_pre_body, prec),
        grid=grid,
        in_specs=[
            pl.BlockSpec((CS, D), lambda i: (i, 0)),
            pl.BlockSpec((1, D), lambda i: (0, 0)),
            pl.BlockSpec((1, D), lambda i: (0, 0)),
            pl.BlockSpec((D, D), lambda i: (0, 0)),
            pl.BlockSpec((D, D), lambda i: (0, 0)),
        ],
        out_specs=pl.BlockSpec((H, CS, 2 * DH), lambda i: (0, i, 0)),
        out_shape=jax.ShapeDtypeStruct((H, S, 2 * DH), jnp.float32),
    )(x2, g.reshape(1, D), b.reshape(1, D), wqk, wv)


def _attn_body(prec, qv_ref, qvp_ref, qps_ref, kpl_ref, kpe_ref, solg_ref):
    for j in range(CB):
        blk = qv_ref[0, j]
        pblk = qvp_ref[0, 0] if j == 0 else qv_ref[0, j - 1]
        q = blk[:, :DH]
        k = jnp.concatenate([pblk[:, :DH], blk[:, :DH]], axis=0)
        kn = k / (jnp.sqrt(jnp.sum(k * k, axis=-1, keepdims=True)) + 1e-9)
        v = jnp.concatenate([pblk[:, DH:], blk[:, DH:]], axis=0)
        qp = qps_ref[0, j]
        kpprev = kpe_ref[0, 0] if j == 0 else kpl_ref[0, j - 1]
        kp = jnp.concatenate([kpprev, kpl_ref[0, j]], axis=-1)
        dots = jax.lax.dot_general(
            q, kn, (((1,), (1,)), ((), ())),
            preferred_element_type=jnp.float32, precision=prec) * 0.125
        dots = jnp.where(qp < kp, -1e9, dots)
        dots = jnp.where(qp == kp, dots - 1e5, dots)
        mx = jnp.max(dots, axis=-1, keepdims=True)
        ex = jnp.exp(dots - mx)
        sm = jnp.sum(ex, axis=-1, keepdims=True)
        num = jax.lax.dot_general(
            ex, v, (((1,), (0,)), ((), ())),
            preferred_element_type=jnp.float32, precision=prec)
        lg = mx + jnp.log(sm)
        solg_ref[0, j] = jnp.concatenate(
            [num / sm, jnp.broadcast_to(lg, (BL, DH))], axis=-1)


def _attn_call(qv4, pos_s, pos_l, prec):
    grid = (NR, NC // CB)
    prev_edge = lambda n, g: (n, (g * CB + NC - 1) % NC, 0, 0)
    return pl.pallas_call(
        functools.partial(_attn_body, prec),
        grid=grid,
        in_specs=[
            pl.BlockSpec((1, CB, BL, 2 * DH), lambda n, g: (n, g, 0, 0)),
            pl.BlockSpec((1, 1, BL, 2 * DH), prev_edge),
            pl.BlockSpec((1, CB, BL, 1), lambda n, g: (n, g, 0, 0)),
            pl.BlockSpec((1, CB, 1, BL), lambda n, g: (n, g, 0, 0)),
            pl.BlockSpec((1, 1, 1, BL), prev_edge),
        ],
        out_specs=pl.BlockSpec((1, CB, BL, 2 * DH), lambda n, g: (n, g, 0, 0)),
        out_shape=jax.ShapeDtypeStruct((NR, NC, BL, 2 * DH), jnp.float32),
    )(qv4, qv4, pos_s, pos_l, pos_l)


CSC = 256


def _comb_body(prec, olg_ref, x1_ref, wo_ref, y1_ref):
    outs = []
    for h in range(H):
        blks = [olg_ref[h * R + r] for r in range(R)]
        lgs = [b[:, DH:DH + 1] for b in blks]
        m = jnp.maximum(jnp.maximum(lgs[0], lgs[1]),
                        jnp.maximum(lgs[2], lgs[3]))
        es = [jnp.exp(l - m) for l in lgs]
        den = es[0] + es[1] + es[2] + es[3]
        acc = blks[0][:, :DH] * (es[0] / den)
        for r in range(1, R):
            acc = acc + blks[r][:, :DH] * (es[r] / den)
        outs.append(acc)
    out = jnp.concatenate(outs, axis=-1)
    y1_ref[...] = x1_ref[...] + _dot(out, wo_ref[...], prec)


def _comb_call(olg_u, x1, wo, prec):
    grid = (S // CSC,)
    return pl.pallas_call(
        functools.partial(_comb_body, prec),
        grid=grid,
        in_specs=[
            pl.BlockSpec((NR, CSC, 2 * DH), lambda i: (0, i, 0)),
            pl.BlockSpec((CSC, D), lambda i: (i, 0)),
            pl.BlockSpec((D, D), lambda i: (0, 0)),
        ],
        out_specs=pl.BlockSpec((CSC, D), lambda i: (i, 0)),
        out_shape=jax.ShapeDtypeStruct((S, D), jnp.float32),
    )(olg_u, x1, wo)


def _ffn_body(prec, y1_ref, x2_ref, g2_ref, b2g_ref, w1_ref, b1_ref, w2_ref,
              b2_ref, y2_ref):
    y1 = y1_ref[...]
    m = jnp.mean(y1, axis=-1, keepdims=True)
    var = jnp.mean((y1 - m) ** 2, axis=-1, keepdims=True)
    ln = (y1 - m) / jnp.sqrt(var + 1e-5) * g2_ref[...] + b2g_ref[...]
    hdn = jnp.maximum(_dot(ln, w1_ref[...], prec) + b1_ref[...], 0.0)
    f = _dot(hdn, w2_ref[...], prec) + b2_ref[...]
    y2_ref[...] = x2_ref[...] + f


def _ffn_call(y1, x2, g2, b2g, w1, b1, w2, b2, prec):
    grid = (S // CS,)
    return pl.pallas_call(
        functools.partial(_ffn_body, prec),
        grid=grid,
        in_specs=[
            pl.BlockSpec((CS, D), lambda i: (i, 0)),
            pl.BlockSpec((CS, D), lambda i: (i, 0)),
            pl.BlockSpec((1, D), lambda i: (0, 0)),
            pl.BlockSpec((1, D), lambda i: (0, 0)),
            pl.BlockSpec((D, DFF), lambda i: (0, 0)),
            pl.BlockSpec((1, DFF), lambda i: (0, 0)),
            pl.BlockSpec((DFF, D), lambda i: (0, 0)),
            pl.BlockSpec((1, D), lambda i: (0, 0)),
        ],
        out_specs=pl.BlockSpec((CS, D), lambda i: (i, 0)),
        out_shape=jax.ShapeDtypeStruct((S, D), jnp.float32),
    )(y1, x2, g2.reshape(1, D), b2g.reshape(1, D),
      w1, b1.reshape(1, DFF), w2, b2.reshape(1, D))


_NW = 32
_GC = 512


def _sc_gather_rows(idx, src):
    def body(idx_ref, src_ref, out_hbm, ivm, rows, sem):
        wid = lax.axis_index("s") * 2 + lax.axis_index("c")
        for t in range(2):
            nr = wid + _NW * t

            @pl.when(nr < NR)
            def _():
                def g_body(g, carry):
                    pltpu.sync_copy(idx_ref.at[nr, pl.ds(g * _GC, _GC)], ivm)
                    pltpu.async_copy(src_ref.at[ivm], rows, sem).wait()
                    pltpu.sync_copy(rows,
                                    out_hbm.at[nr, pl.ds(g * _GC, _GC)])
                    return carry

                lax.fori_loop(0, S // _GC, g_body, 0)

    f = pl.kernel(
        body,
        mesh=plsc.VectorSubcoreMesh(core_axis_name="c", subcore_axis_name="s"),
        out_type=jax.ShapeDtypeStruct((NR, S, 2 * DH), jnp.float32),
        scratch_types=[
            pltpu.VMEM((_GC,), jnp.int32),
            pltpu.VMEM((_GC, 2 * DH), jnp.float32),
            pltpu.SemaphoreType.DMA,
        ],
    )
    return f(idx, src)


def _buckets_for_layer(x2, g, b, wqk, rot):
    xm = x2.mean(-1, keepdims=True)
    xv = ((x2 - xm) ** 2).mean(-1, keepdims=True)
    x = (x2 - xm) / jnp.sqrt(xv + 1e-5) * g + b
    qk = (x @ wqk).reshape(S, H, DH).transpose(1, 0, 2)
    rotated = jnp.einsum('nsd,rdb->nrsb', qk, rot)
    rotated = jnp.concatenate([rotated, -rotated], axis=-1)
    return jnp.argmax(rotated, axis=-1).reshape(NR, S)


def _layer(x1, x2, wqk, wv, wo, g1, b1g, w1, b1, w2, b2, g2, b2g, rot, last):
    prec = jax.lax.Precision.DEFAULT if last else jax.lax.Precision.HIGHEST
    qv_h = _pre_call(x2, g1, b1g, wqk, wv, prec)

    buckets = _buckets_for_layer(x2, g1, b1g, wqk, rot)
    tick = jnp.arange(S)
    sticker = jnp.argsort(buckets * S + tick[None, :], axis=-1)
    undo = jnp.argsort(sticker, axis=-1)
    row_off = (jnp.arange(NR, dtype=jnp.int32) // R * S)[:, None]

    sqv = _sc_gather_rows(sticker + row_off, qv_h.reshape(N * S, 2 * DH))

    qv4 = sqv.reshape(NR, NC, BL, 2 * DH)
    stick3 = sticker.reshape(NR, NC, BL).astype(jnp.float32)
    pos_s = stick3[..., None]
    pos_l = stick3[:, :, None, :]

    solg4 = _attn_call(qv4, pos_s, pos_l, prec)
    row_off2 = (jnp.arange(NR, dtype=jnp.int32) * S)[:, None]
    olg_u = _sc_gather_rows(undo + row_off2,
                            solg4.reshape(NR * S, 2 * DH))

    y1 = _comb_call(olg_u, x1, wo, prec)
    y2 = _ffn_call(y1, x2, g2, b2g, w1, b1, w2, b2, prec)
    return y1, y2


def kernel(x1, x2, mask, Wqk, Wv, Wo, ln1_g, ln1_b, W1, b1, W2, b2,
           ln2_g, ln2_b, rot):
    a1 = x1[0]
    a2 = x2[0]
    for l in range(L):
        y1, y2 = _layer(a1, a2, Wqk[l], Wv[l], Wo[l], ln1_g[l], ln1_b[l],
                        W1[l], b1[l], W2[l], b2[l], ln2_g[l], ln2_b[l],
                        rot[l], last=(l == L - 1))
        a1, a2 = y1, y2
    return a2[None]


# speedup vs baseline: 2.6755x; 1.0372x over previous
import functools
import jax
import jax.numpy as jnp
from jax import lax
from jax.experimental import pallas as pl
from jax.experimental.pallas import tpu as pltpu
from jax.experimental.pallas import tpu_sc as plsc

B, S, D, H, DH, L, R, BL, DFF = 1, 4096, 768, 12, 64, 2, 4, 64, 3072
NB = S // BL
NC = S // BL
N = B * H
NR = N * R
CS = 512
CB = 4

_PREC = jax.lax.Precision.HIGHEST


def _dot(a, b, prec=_PREC):
    if prec is jax.lax.Precision.HIGHEST:
        ah = a.astype(jnp.bfloat16)
        al = (a - ah.astype(jnp.float32)).astype(jnp.bfloat16)
        bh = b.astype(jnp.bfloat16)
        bl = (b - bh.astype(jnp.float32)).astype(jnp.bfloat16)

        def d(x, y):
            return jnp.dot(x, y, preferred_element_type=jnp.float32)

        return d(ah, bh) + (d(ah, bl) + d(al, bh))
    return jnp.dot(a, b, preferred_element_type=jnp.float32, precision=prec)


def _pre_body(prec, x_ref, g_ref, b_ref, wqk_ref, wv_ref, qv_ref):
    x = x_ref[...]
    m = jnp.mean(x, axis=-1, keepdims=True)
    var = jnp.mean((x - m) ** 2, axis=-1, keepdims=True)
    ln = (x - m) / jnp.sqrt(var + 1e-5) * g_ref[...] + b_ref[...]
    qk = _dot(ln, wqk_ref[...], prec)
    v = _dot(ln, wv_ref[...], prec)
    for h in range(H):
        qv_ref[h, :, :DH] = qk[:, h * DH:(h + 1) * DH]
        qv_ref[h, :, DH:] = v[:, h * DH:(h + 1) * DH]


def _pre_call(x2, g, b, wqk, wv, prec):
    grid = (S // CS,)
    return pl.pallas_call(
        functools.partial(_pre_body, prec),
        grid=grid,
        in_specs=[
            pl.BlockSpec((CS, D), lambda i: (i, 0)),
            pl.BlockSpec((1, D), lambda i: (0, 0)),
            pl.BlockSpec((1, D), lambda i: (0, 0)),
            pl.BlockSpec((D, D), lambda i: (0, 0)),
            pl.BlockSpec((D, D), lambda i: (0, 0)),
        ],
        out_specs=pl.BlockSpec((H, CS, 2 * DH), lambda i: (0, i, 0)),
        out_shape=jax.ShapeDtypeStruct((H, S, 2 * DH), jnp.float32),
    )(x2, g.reshape(1, D), b.reshape(1, D), wqk, wv)


def _attn_body(prec, qv_ref, qvp_ref, qps_ref, kpl_ref, kpe_ref, solg_ref):
    for j in range(CB):
        blk = qv_ref[0, j]
        pblk = qvp_ref[0, 0] if j == 0 else qv_ref[0, j - 1]
        q = blk[:, :DH]
        k = jnp.concatenate([pblk[:, :DH], blk[:, :DH]], axis=0)
        kn = k / (jnp.sqrt(jnp.sum(k * k, axis=-1, keepdims=True)) + 1e-9)
        v = jnp.concatenate([pblk[:, DH:], blk[:, DH:]], axis=0)
        qp = qps_ref[0, j]
        kpprev = kpe_ref[0, 0] if j == 0 else kpl_ref[0, j - 1]
        kp = jnp.concatenate([kpprev, kpl_ref[0, j]], axis=-1)
        dots = jax.lax.dot_general(
            q, kn, (((1,), (1,)), ((), ())),
            preferred_element_type=jnp.float32, precision=prec) * 0.125
        dots = jnp.where(qp < kp, -1e9, dots)
        dots = jnp.where(qp == kp, dots - 1e5, dots)
        mx = jnp.max(dots, axis=-1, keepdims=True)
        ex = jnp.exp(dots - mx)
        sm = jnp.sum(ex, axis=-1, keepdims=True)
        num = jax.lax.dot_general(
            ex, v, (((1,), (0,)), ((), ())),
            preferred_element_type=jnp.float32, precision=prec)
        lg = mx + jnp.log(sm)
        solg_ref[0, j] = jnp.concatenate(
            [num / sm, jnp.broadcast_to(lg, (BL, DH))], axis=-1)


def _attn_call(qv4, pos_s, pos_l, prec):
    grid = (NR, NC // CB)
    prev_edge = lambda n, g: (n, (g * CB + NC - 1) % NC, 0, 0)
    return pl.pallas_call(
        functools.partial(_attn_body, prec),
        grid=grid,
        in_specs=[
            pl.BlockSpec((1, CB, BL, 2 * DH), lambda n, g: (n, g, 0, 0)),
            pl.BlockSpec((1, 1, BL, 2 * DH), prev_edge),
            pl.BlockSpec((1, CB, BL, 1), lambda n, g: (n, g, 0, 0)),
            pl.BlockSpec((1, CB, 1, BL), lambda n, g: (n, g, 0, 0)),
            pl.BlockSpec((1, 1, 1, BL), prev_edge),
        ],
        out_specs=pl.BlockSpec((1, CB, BL, 2 * DH), lambda n, g: (n, g, 0, 0)),
        out_shape=jax.ShapeDtypeStruct((NR, NC, BL, 2 * DH), jnp.float32),
    )(qv4, qv4, pos_s, pos_l, pos_l)


CSC = 256


def _comb_body(prec, olg_ref, x1_ref, wo_ref, y1_ref):
    outs = []
    for h in range(H):
        blks = [olg_ref[h * R + r] for r in range(R)]
        lgs = [b[:, DH:DH + 1] for b in blks]
        m = jnp.maximum(jnp.maximum(lgs[0], lgs[1]),
                        jnp.maximum(lgs[2], lgs[3]))
        es = [jnp.exp(l - m) for l in lgs]
        den = es[0] + es[1] + es[2] + es[3]
        acc = blks[0][:, :DH] * (es[0] / den)
        for r in range(1, R):
            acc = acc + blks[r][:, :DH] * (es[r] / den)
        outs.append(acc)
    out = jnp.concatenate(outs, axis=-1)
    y1_ref[...] = x1_ref[...] + _dot(out, wo_ref[...], prec)


def _comb_call(olg_u, x1, wo, prec):
    grid = (S // CSC,)
    return pl.pallas_call(
        functools.partial(_comb_body, prec),
        grid=grid,
        in_specs=[
            pl.BlockSpec((NR, CSC, 2 * DH), lambda i: (0, i, 0)),
            pl.BlockSpec((CSC, D), lambda i: (i, 0)),
            pl.BlockSpec((D, D), lambda i: (0, 0)),
        ],
        out_specs=pl.BlockSpec((CSC, D), lambda i: (i, 0)),
        out_shape=jax.ShapeDtypeStruct((S, D), jnp.float32),
    )(olg_u, x1, wo)


def _ffn_body(prec, y1_ref, x2_ref, g2_ref, b2g_ref, w1_ref, b1_ref, w2_ref,
              b2_ref, y2_ref):
    y1 = y1_ref[...]
    m = jnp.mean(y1, axis=-1, keepdims=True)
    var = jnp.mean((y1 - m) ** 2, axis=-1, keepdims=True)
    ln = (y1 - m) / jnp.sqrt(var + 1e-5) * g2_ref[...] + b2g_ref[...]
    hdn = jnp.maximum(_dot(ln, w1_ref[...], prec) + b1_ref[...], 0.0)
    f = _dot(hdn, w2_ref[...], prec) + b2_ref[...]
    y2_ref[...] = x2_ref[...] + f


def _ffn_call(y1, x2, g2, b2g, w1, b1, w2, b2, prec):
    grid = (S // CS,)
    return pl.pallas_call(
        functools.partial(_ffn_body, prec),
        grid=grid,
        in_specs=[
            pl.BlockSpec((CS, D), lambda i: (i, 0)),
            pl.BlockSpec((CS, D), lambda i: (i, 0)),
            pl.BlockSpec((1, D), lambda i: (0, 0)),
            pl.BlockSpec((1, D), lambda i: (0, 0)),
            pl.BlockSpec((D, DFF), lambda i: (0, 0)),
            pl.BlockSpec((1, DFF), lambda i: (0, 0)),
            pl.BlockSpec((DFF, D), lambda i: (0, 0)),
            pl.BlockSpec((1, D), lambda i: (0, 0)),
        ],
        out_specs=pl.BlockSpec((CS, D), lambda i: (i, 0)),
        out_shape=jax.ShapeDtypeStruct((S, D), jnp.float32),
    )(y1, x2, g2.reshape(1, D), b2g.reshape(1, D),
      w1, b1.reshape(1, DFF), w2, b2.reshape(1, D))


_NW = 32
_GC = 512


def _sc_gather_rows(idx, src):
    def body(idx_ref, src_ref, out_hbm, ivm, rows, sem):
        wid = lax.axis_index("s") * 2 + lax.axis_index("c")
        for t in range(2):
            nr = wid + _NW * t

            @pl.when(nr < NR)
            def _():
                def g_body(g, carry):
                    pltpu.sync_copy(idx_ref.at[nr, pl.ds(g * _GC, _GC)], ivm)
                    pltpu.async_copy(src_ref.at[ivm], rows, sem).wait()
                    pltpu.sync_copy(rows,
                                    out_hbm.at[nr, pl.ds(g * _GC, _GC)])
                    return carry

                lax.fori_loop(0, S // _GC, g_body, 0)

    f = pl.kernel(
        body,
        mesh=plsc.VectorSubcoreMesh(core_axis_name="c", subcore_axis_name="s"),
        out_type=jax.ShapeDtypeStruct((NR, S, 2 * DH), jnp.float32),
        scratch_types=[
            pltpu.VMEM((_GC,), jnp.int32),
            pltpu.VMEM((_GC, 2 * DH), jnp.float32),
            pltpu.SemaphoreType.DMA,
        ],
    )
    return f(idx, src)


def _buckets_for_layer(x2, g, b, wqk, rot):
    xm = x2.mean(-1, keepdims=True)
    xv = ((x2 - xm) ** 2).mean(-1, keepdims=True)
    x = (x2 - xm) / jnp.sqrt(xv + 1e-5) * g + b
    qk = (x @ wqk).reshape(S, H, DH).transpose(1, 0, 2)
    rotated = jnp.einsum('nsd,rdb->nrsb', qk, rot)
    rotated = jnp.concatenate([rotated, -rotated], axis=-1)
    return jnp.argmax(rotated, axis=-1).reshape(NR, S)


def _layer(x1, x2, wqk, wv, wo, g1, b1g, w1, b1, w2, b2, g2, b2g, rot, last):
    prec = jax.lax.Precision.DEFAULT if last else jax.lax.Precision.HIGHEST
    qv_h = _pre_call(x2, g1, b1g, wqk, wv, prec)

    buckets = _buckets_for_layer(x2, g1, b1g, wqk, rot)
    tick = jnp.arange(S)
    sticker = jnp.argsort(buckets * S + tick[None, :], axis=-1)
    undo = jnp.argsort(sticker, axis=-1)
    row_off = (jnp.arange(NR, dtype=jnp.int32) // R * S)[:, None]

    sqv = _sc_gather_rows(sticker + row_off, qv_h.reshape(N * S, 2 * DH))

    qv4 = sqv.reshape(NR, NC, BL, 2 * DH)
    stick3 = sticker.reshape(NR, NC, BL).astype(jnp.float32)
    pos_s = stick3[..., None]
    pos_l = stick3[:, :, None, :]

    solg4 = _attn_call(qv4, pos_s, pos_l, prec)
    row_off2 = (jnp.arange(NR, dtype=jnp.int32) * S)[:, None]
    olg_u = _sc_gather_rows(undo + row_off2,
                            solg4.reshape(NR * S, 2 * DH))

    y1 = _comb_call(olg_u, x1, wo, prec)
    y2 = _ffn_call(y1, x2, g2, b2g, w1, b1, w2, b2, prec)
    return y1, y2


def kernel(x1, x2, mask, Wqk, Wv, Wo, ln1_g, ln1_b, W1, b1, W2, b2,
           ln2_g, ln2_b, rot):
    a1 = x1[0]
    a2 = x2[0]
    for l in range(L):
        y1, y2 = _layer(a1, a2, Wqk[l], Wv[l], Wo[l], ln1_g[l], ln1_b[l],
                        W1[l], b1[l], W2[l], b2[l], ln2_g[l], ln2_b[l],
                        rot[l], last=(l == L - 1))
        a1, a2 = y1, y2
    return a2[None]
